# baseline (speedup 1.0000x reference)
import functools

import jax
import jax.numpy as jnp
from jax import lax
from jax.experimental import pallas as pl
from jax.experimental.pallas import tpu as pltpu
from jax.experimental.pallas import tpu_sc as plsc

SEQ = 200
D = 128
BATCH = 4096
NUM_WORKERS = 32
ROWS_PER_W = BATCH // NUM_WORKERS
CH_A = 104
CH_B = SEQ - CH_A
NBUF = 3

_mesh = plsc.VectorSubcoreMesh(core_axis_name="c", subcore_axis_name="s")


@functools.partial(
    pl.kernel,
    out_type=jax.ShapeDtypeStruct((BATCH * SEQ, D), jnp.float32),
    mesh=_mesh,
    scratch_types=[
        pltpu.VMEM((SEQ, D), jnp.float32),
        pltpu.VMEM((ROWS_PER_W * SEQ,), jnp.int32),
        pltpu.VMEM((NBUF, SEQ, D), jnp.float32),
        pltpu.SemaphoreType.DMA,
        [pltpu.SemaphoreType.DMA] * NBUF,
    ],
)
def _emb(idx_hbm, tok_hbm, pos_hbm, out_hbm, pos_v, idx_v, rows_v, sem_g, sem_o):
    wid = lax.axis_index("s") * 2 + lax.axis_index("c")
    wbase = wid * ROWS_PER_W * SEQ

    pltpu.sync_copy(pos_hbm, pos_v)
    pltpu.sync_copy(idx_hbm.at[pl.ds(wbase, ROWS_PER_W * SEQ)], idx_v)

    def fill_pos(b):
        @pl.loop(0, SEQ, unroll=4)
        def _fill(l):
            for j in range(D // 16):
                sl = pl.ds(j * 16, 16)
                rows_v[b, l, sl] = pos_v[l, sl]

    def issue_gather(r, b):
        off = r * SEQ
        pltpu.async_copy(
            tok_hbm.at[idx_v.at[pl.ds(off, CH_A)]],
            rows_v.at[b, pl.ds(0, CH_A)], sem_g, add=True)
        pltpu.async_copy(
            tok_hbm.at[idx_v.at[pl.ds(off + CH_A, CH_B)]],
            rows_v.at[b, pl.ds(CH_A, CH_B)], sem_g, add=True)

    def wait_gather(b):
        pltpu.make_async_copy(
            tok_hbm.at[idx_v.at[pl.ds(0, CH_A)]],
            rows_v.at[b, pl.ds(0, CH_A)], sem_g).wait()
        pltpu.make_async_copy(
            tok_hbm.at[idx_v.at[pl.ds(0, CH_B)]],
            rows_v.at[b, pl.ds(CH_A, CH_B)], sem_g).wait()

    def issue_out(r, b):
        pltpu.async_copy(rows_v.at[b], out_hbm.at[pl.ds(wbase + r * SEQ, SEQ)], sem_o[b])

    def wait_out(b):
        pltpu.make_async_copy(rows_v.at[b], out_hbm.at[pl.ds(wbase, SEQ)], sem_o[b]).wait()

    def body(r, b, wait_o=True, gather=True):
        wait_gather(b)
        issue_out(r, b)
        if gather:
            b2 = (b + 2) % NBUF
            if wait_o:
                wait_out(b2)
            fill_pos(b2)
            issue_gather(r + 2, b2)

    fill_pos(0)
    fill_pos(1)
    issue_gather(0, 0)
    issue_gather(1, 1)

    body(0, 0, wait_o=False)
    body(1, 1)
    body(2, 2)

    @pl.loop(NBUF, ROWS_PER_W - 2, step=NBUF)
    def _ring(r0):
        for k in range(NBUF):
            body(r0 + k, k)

    body(ROWS_PER_W - 2, 0, gather=False)
    body(ROWS_PER_W - 1, 1, gather=False)

    for b in range(NBUF):
        wait_out(b)


def kernel(inputs, token_table, pos_table):
    b, l = inputs.shape
    flat_idx = inputs.reshape(b * l)
    out = _emb(flat_idx, token_table, pos_table)
    return out.reshape(b, l, token_table.shape[1])


# speedup vs baseline: 1.3339x; 1.3339x over previous
import functools

import jax
import jax.numpy as jnp
from jax import lax
from jax.experimental import pallas as pl
from jax.experimental.pallas import tpu as pltpu
from jax.experimental.pallas import tpu_sc as plsc

SEQ = 200
D = 128
BATCH = 4096
NUM_WORKERS = 32
ROWS_PER_W = BATCH // NUM_WORKERS
CH_A = 128
CH_B = SEQ - CH_A
NBUF = 5

_mesh = plsc.VectorSubcoreMesh(core_axis_name="c", subcore_axis_name="s")


@functools.partial(
    pl.kernel,
    out_type=jax.ShapeDtypeStruct((BATCH * SEQ, D), jnp.float32),
    mesh=_mesh,
    scratch_types=[
        pltpu.VMEM_SHARED((SEQ, D), jnp.float32),
        [pltpu.VMEM((SEQ,), jnp.int32)] * NBUF,
        pltpu.VMEM((NBUF, SEQ, D), jnp.float32),
        pltpu.SemaphoreType.DMA,
        [pltpu.SemaphoreType.DMA] * NBUF,
        [pltpu.SemaphoreType.DMA] * NBUF,
        [pltpu.SemaphoreType.DMA] * NBUF,
    ],
)
def _emb(idx_hbm, tok_hbm, pos_hbm, out_hbm, pos_sh, idx_v, rows_v,
         sem_g, sem_o, sem_p, sem_i):
    wid = lax.axis_index("s") * 2 + lax.axis_index("c")
    wbase = wid * ROWS_PER_W * SEQ

    @pl.when(lax.axis_index("s") == 0)
    def _seed():
        pltpu.sync_copy(pos_hbm, rows_v.at[0])
        pltpu.sync_copy(rows_v.at[0], pos_sh)

    plsc.subcore_barrier()

    def issue_fetch(r, b):
        pltpu.async_copy(idx_hbm.at[pl.ds(wbase + r * SEQ, SEQ)], idx_v[b], sem_i[b])

    def wait_fetch(b):
        pltpu.make_async_copy(idx_hbm.at[pl.ds(wbase, SEQ)], idx_v[b], sem_i[b]).wait()

    def issue_prefill(b):
        pltpu.async_copy(pos_sh, rows_v.at[b], sem_p[b])

    def wait_prefill(b):
        pltpu.make_async_copy(pos_sh, rows_v.at[b], sem_p[b]).wait()

    def issue_gather(b):
        pltpu.async_copy(
            tok_hbm.at[idx_v[b].at[pl.ds(0, CH_A)]],
            rows_v.at[b, pl.ds(0, CH_A)], sem_g, add=True)
        pltpu.async_copy(
            tok_hbm.at[idx_v[b].at[pl.ds(CH_A, CH_B)]],
            rows_v.at[b, pl.ds(CH_A, CH_B)], sem_g, add=True)

    def wait_gather(b):
        pltpu.make_async_copy(
            tok_hbm.at[idx_v[b].at[pl.ds(0, CH_A)]],
            rows_v.at[b, pl.ds(0, CH_A)], sem_g).wait()
        pltpu.make_async_copy(
            tok_hbm.at[idx_v[b].at[pl.ds(0, CH_B)]],
            rows_v.at[b, pl.ds(CH_A, CH_B)], sem_g).wait()

    def issue_out(r, b):
        pltpu.async_copy(rows_v.at[b], out_hbm.at[pl.ds(wbase + r * SEQ, SEQ)], sem_o[b])

    def wait_out(b):
        pltpu.make_async_copy(rows_v.at[b], out_hbm.at[pl.ds(wbase, SEQ)], sem_o[b]).wait()

    def body(r, b, wait_o=True, prefill=True, gather=True):
        wait_gather(b)
        issue_out(r, b)
        if prefill:
            b4 = (b + 4) % NBUF
            if wait_o:
                wait_out(b4)
            issue_prefill(b4)
            issue_fetch(r + 4, b4)
        if gather:
            b3 = (b + 3) % NBUF
            wait_prefill(b3)
            wait_fetch(b3)
            issue_gather(b3)

    for b in range(4):
        issue_prefill(b)
        issue_fetch(b, b)
    for r in range(3):
        wait_prefill(r)
        wait_fetch(r)
        issue_gather(r)

    body(0, 0, wait_o=False)
    body(1, 1)
    body(2, 2)
    body(3, 3)
    body(4, 4)

    @pl.loop(NBUF, ROWS_PER_W - 8, step=NBUF)
    def _ring(r0):
        for k in range(NBUF):
            body(r0 + k, k)

    body(ROWS_PER_W - 8, 0)
    body(ROWS_PER_W - 7, 1)
    body(ROWS_PER_W - 6, 2)
    body(ROWS_PER_W - 5, 3)
    body(ROWS_PER_W - 4, 4, prefill=False)
    body(ROWS_PER_W - 3, 0, prefill=False, gather=False)
    body(ROWS_PER_W - 2, 1, prefill=False, gather=False)
    body(ROWS_PER_W - 1, 2, prefill=False, gather=False)

    for b in range(NBUF):
        wait_out(b)


def kernel(inputs, token_table, pos_table):
    b, l = inputs.shape
    flat_idx = inputs.reshape(b * l)
    out = _emb(flat_idx, token_table, pos_table)
    return out.reshape(b, l, token_table.shape[1])


# speedup vs baseline: 1.3371x; 1.0024x over previous
import functools

import jax
import jax.numpy as jnp
from jax import lax
from jax.experimental import pallas as pl
from jax.experimental.pallas import tpu as pltpu
from jax.experimental.pallas import tpu_sc as plsc

SEQ = 200
D = 128
BATCH = 4096
NUM_WORKERS = 32
ROWS_PER_W = BATCH // NUM_WORKERS
CH_A = 128
CH_B = SEQ - CH_A
NBUF = 5

_mesh = plsc.VectorSubcoreMesh(core_axis_name="c", subcore_axis_name="s")


@functools.partial(
    pl.kernel,
    out_type=jax.ShapeDtypeStruct((BATCH * SEQ, D), jnp.float32),
    mesh=_mesh,
    scratch_types=[
        pltpu.VMEM_SHARED((SEQ, D), jnp.float32),
        [pltpu.VMEM((SEQ,), jnp.int32)] * NBUF,
        pltpu.VMEM((NBUF, SEQ, D), jnp.float32),
        pltpu.SemaphoreType.DMA,
        [pltpu.SemaphoreType.DMA] * NBUF,
        [pltpu.SemaphoreType.DMA] * NBUF,
        [pltpu.SemaphoreType.DMA] * NBUF,
    ],
)
def _emb(idx_hbm, tok_hbm, pos_hbm, out_hbm, pos_sh, idx_v, rows_v,
         sem_g, sem_o, sem_p, sem_i):
    wid = lax.axis_index("s") * 2 + lax.axis_index("c")
    wbase = wid * ROWS_PER_W * SEQ

    @pl.when(lax.axis_index("s") == 0)
    def _seed():
        pltpu.sync_copy(pos_hbm, rows_v.at[0])
        pltpu.sync_copy(rows_v.at[0], pos_sh)

    plsc.subcore_barrier()

    def issue_fetch(r, b):
        pltpu.async_copy(idx_hbm.at[pl.ds(wbase + r * SEQ, SEQ)], idx_v[b], sem_i[b])

    def wait_fetch(b):
        pltpu.make_async_copy(idx_hbm.at[pl.ds(wbase, SEQ)], idx_v[b], sem_i[b]).wait()

    def issue_prefill(b):
        pltpu.async_copy(pos_sh, rows_v.at[b], sem_p[b])

    def wait_prefill(b):
        pltpu.make_async_copy(pos_sh, rows_v.at[b], sem_p[b]).wait()

    def issue_gather(b):
        pltpu.async_copy(
            tok_hbm.at[idx_v[b].at[pl.ds(0, CH_A)]],
            rows_v.at[b, pl.ds(0, CH_A)], sem_g, add=True)
        pltpu.async_copy(
            tok_hbm.at[idx_v[b].at[pl.ds(CH_A, CH_B)]],
            rows_v.at[b, pl.ds(CH_A, CH_B)], sem_g, add=True)

    def wait_gather(b):
        pltpu.make_async_copy(
            tok_hbm.at[idx_v[b].at[pl.ds(0, CH_A)]],
            rows_v.at[b, pl.ds(0, CH_A)], sem_g).wait()
        pltpu.make_async_copy(
            tok_hbm.at[idx_v[b].at[pl.ds(0, CH_B)]],
            rows_v.at[b, pl.ds(CH_A, CH_B)], sem_g).wait()

    def issue_out(r, b):
        pltpu.async_copy(rows_v.at[b], out_hbm.at[pl.ds(wbase + r * SEQ, SEQ)], sem_o[b])

    def wait_out(b):
        pltpu.make_async_copy(rows_v.at[b], out_hbm.at[pl.ds(wbase, SEQ)], sem_o[b]).wait()

    def body(r, b, wait_o=True, prefill=True, gather=True):
        wait_gather(b)
        issue_out(r, b)
        if gather:
            b3 = (b + 3) % NBUF
            wait_prefill(b3)
            wait_fetch(b3)
            issue_gather(b3)
        if prefill:
            b4 = (b + 4) % NBUF
            if wait_o:
                wait_out(b4)
            issue_prefill(b4)
            issue_fetch(r + 4, b4)

    for b in range(4):
        issue_prefill(b)
        issue_fetch(b, b)
    for r in range(3):
        wait_prefill(r)
        wait_fetch(r)
        issue_gather(r)

    body(0, 0, wait_o=False)
    body(1, 1)
    body(2, 2)
    body(3, 3)
    body(4, 4)

    @pl.loop(NBUF, ROWS_PER_W - 8, step=NBUF)
    def _ring(r0):
        for k in range(NBUF):
            body(r0 + k, k)

    body(ROWS_PER_W - 8, 0)
    body(ROWS_PER_W - 7, 1)
    body(ROWS_PER_W - 6, 2)
    body(ROWS_PER_W - 5, 3)
    body(ROWS_PER_W - 4, 4, prefill=False)
    body(ROWS_PER_W - 3, 0, prefill=False, gather=False)
    body(ROWS_PER_W - 2, 1, prefill=False, gather=False)
    body(ROWS_PER_W - 1, 2, prefill=False, gather=False)

    for b in range(NBUF):
        wait_out(b)


def kernel(inputs, token_table, pos_table):
    b, l = inputs.shape
    flat_idx = inputs.reshape(b * l)
    out = _emb(flat_idx, token_table, pos_table)
    return out.reshape(b, l, token_table.shape[1])
